# baseline (speedup 1.0000x reference)
import jax
import jax.numpy as jnp
from jax.experimental import pallas as pl
from jax.experimental.pallas import tpu as pltpu

D_MODEL = 256
ROW_BLOCK = 8336


def _proj_kernel(ctx_ref, smp_ref, w_ref, b_ref, out_ref):
    h = pl.program_id(1)
    w = w_ref[...]
    b = b_ref[...]

    @pl.when(h == 0)
    def _():
        out_ref[0] = jnp.dot(ctx_ref[...], w, preferred_element_type=jnp.float32) + b

    @pl.when(h == 1)
    def _():
        out_ref[0] = jnp.dot(smp_ref[...], w, preferred_element_type=jnp.float32) + b


def kernel(context, sample, W_proj, b_proj):
    n, d = context.shape
    blk = min(ROW_BLOCK, n)
    nb = pl.cdiv(n, blk)
    b2 = b_proj.reshape(1, d)
    out = pl.pallas_call(
        _proj_kernel,
        grid=(nb, 2),
        in_specs=[
            pl.BlockSpec((blk, d), lambda i, h: (i, 0)),
            pl.BlockSpec((blk, d), lambda i, h: (i, 0)),
            pl.BlockSpec((d, d), lambda i, h: (0, 0)),
            pl.BlockSpec((1, d), lambda i, h: (0, 0)),
        ],
        out_specs=pl.BlockSpec((1, blk, d), lambda i, h: (h, i, 0)),
        out_shape=jax.ShapeDtypeStruct((2, n, d), jnp.float32),
        compiler_params=pltpu.CompilerParams(
            dimension_semantics=("arbitrary", "arbitrary"),
            vmem_limit_bytes=100 * 1024 * 1024,
        ),
    )(context, sample, W_proj, b2)
    return out.reshape(2 * n, d)


# speedup vs baseline: 1.2824x; 1.2824x over previous
import jax
import jax.numpy as jnp
from jax.experimental import pallas as pl
from jax.experimental.pallas import tpu as pltpu

D_MODEL = 256
ROW_BLOCK = 8144


def _proj_kernel(ctx_ref, smp_ref, w_ref, b_ref, out_ref):
    w = w_ref[...]
    b = b_ref[...]
    out_ref[0] = jnp.dot(ctx_ref[...], w, preferred_element_type=jnp.float32) + b
    out_ref[1] = jnp.dot(smp_ref[...], w, preferred_element_type=jnp.float32) + b


def kernel(context, sample, W_proj, b_proj):
    n, d = context.shape
    blk = min(ROW_BLOCK, n)
    nb = pl.cdiv(n, blk)
    b2 = b_proj.reshape(1, d)
    out = pl.pallas_call(
        _proj_kernel,
        grid=(nb,),
        in_specs=[
            pl.BlockSpec((blk, d), lambda i: (i, 0)),
            pl.BlockSpec((blk, d), lambda i: (i, 0)),
            pl.BlockSpec((d, d), lambda i: (0, 0)),
            pl.BlockSpec((1, d), lambda i: (0, 0)),
        ],
        out_specs=pl.BlockSpec((2, blk, d), lambda i: (0, i, 0)),
        out_shape=jax.ShapeDtypeStruct((2, n, d), jnp.float32),
        compiler_params=pltpu.CompilerParams(
            dimension_semantics=("parallel",),
            vmem_limit_bytes=100 * 1024 * 1024,
        ),
    )(context, sample, W_proj, b2)
    return out.reshape(2 * n, d)
